# baseline (speedup 1.0000x reference)
import functools

import jax
import jax.numpy as jnp
from jax import lax
from jax.experimental import pallas as pl
from jax.experimental.pallas import tpu as pltpu
from jax.experimental.pallas import tpu_sc as plsc

_N = 10000
_E = 160000
_D = 256
_DH = _D // 2

_NT = 16
_CH = 40
_NCH = 250
_EPT = _NCH * _CH

_ACC_ROWS = 10112
_ZROWS = _ACC_ROWS // _NT

_mesh = plsc.VectorSubcoreMesh(core_axis_name="c", subcore_axis_name="s")


@functools.partial(
    pl.kernel,
    out_type=jax.ShapeDtypeStruct((2, _ACC_ROWS, _DH), jnp.float32),
    mesh=_mesh,
    scratch_types=[
        pltpu.VMEM((_CH,), jnp.int32),
        pltpu.VMEM((_CH,), jnp.int32),
        pltpu.VMEM((_CH,), jnp.int32),
        pltpu.VMEM((_CH,), jnp.int32),
        pltpu.VMEM((_CH,), jnp.int32),
        pltpu.VMEM((_CH,), jnp.int32),
        pltpu.VMEM((_CH, _DH), jnp.float32),
        pltpu.VMEM((_CH, _DH), jnp.float32),
        pltpu.VMEM((_CH, _DH), jnp.float32),
        pltpu.VMEM_SHARED((_ACC_ROWS, _DH), jnp.float32),
        pltpu.SemaphoreType.DMA,
        pltpu.SemaphoreType.DMA,
        pltpu.SemaphoreType.DMA,
        pltpu.SemaphoreType.DMA,
        pltpu.SemaphoreType.DMA,
        pltpu.SemaphoreType.DMA,
        pltpu.SemaphoreType.DMA,
        pltpu.SemaphoreType.DMA,
        pltpu.SemaphoreType.DMA,
        pltpu.SemaphoreType.DMA,
        pltpu.SemaphoreType.DMA,
        pltpu.SemaphoreType.DMA,
    ],
)
def _sc_neigh(src_hbm, dst_hbm, h0_hbm, h1_hbm, zeros_hbm, out_hbm,
              srcA, srcB, srcC, dstA, dstB, dstC, rows0, rows1, rows2, acc,
              semSrcA, semSrcB, semSrcC, semDstA, semDstB, semDstC,
              semG0, semG1, semG2, semS0, semS1, semS2):
    c = lax.axis_index("c")
    s = lax.axis_index("s")
    srcI = (srcA, srcB, srcC)
    dstI = (dstA, dstB, dstC)
    semSrc = (semSrcA, semSrcB, semSrcC)
    semDst = (semDstA, semDstB, semDstC)
    rowsR = (rows0, rows1, rows2)
    semG = (semG0, semG1, semG2)
    semS = (semS0, semS1, semS2)

    pltpu.sync_copy(zeros_hbm, acc.at[pl.ds(s * _ZROWS, _ZROWS)])
    plsc.subcore_barrier()

    def src_load(j, k):
        pltpu.async_copy(src_hbm.at[pl.ds(s * _EPT + j * _CH, _CH)],
                         srcI[k], semSrc[k])

    def src_wait(j, k):
        pltpu.make_async_copy(src_hbm.at[pl.ds(s * _EPT + j * _CH, _CH)],
                              srcI[k], semSrc[k]).wait()

    def dst_load(j, k):
        pltpu.async_copy(dst_hbm.at[pl.ds(s * _EPT + j * _CH, _CH)],
                         dstI[k], semDst[k])

    def dst_wait(j, k):
        pltpu.make_async_copy(dst_hbm.at[pl.ds(s * _EPT + j * _CH, _CH)],
                              dstI[k], semDst[k]).wait()

    def run(h_ref):
        def gather(k):
            pltpu.async_copy(h_ref.at[srcI[k]], rowsR[k], semG[k])

        def gather_wait(k):
            pltpu.make_async_copy(h_ref.at[srcI[k]], rowsR[k], semG[k]).wait()

        def scatter(k):
            pltpu.async_copy(rowsR[k], acc.at[dstI[k]], semS[k], add=True)

        def scatter_wait(k):
            pltpu.make_async_copy(rowsR[k], acc.at[dstI[k]], semS[k]).wait()

        def step(jj, k, do_scatter_wait=True, do_dst_load=True,
                 do_gather_next=True, do_src_load=True):
            kn = (k + 1) % 3
            if do_scatter_wait:
                scatter_wait(kn)
            if do_dst_load:
                dst_load(jj + 1, kn)
            if do_gather_next:
                src_wait(jj + 1, kn)
                gather(kn)
            gather_wait(k)
            if do_src_load:
                src_load(jj + 3, k)
            dst_wait(jj, k)
            scatter(k)

        src_load(0, 0)
        dst_load(0, 0)
        src_load(1, 1)
        dst_load(1, 1)
        src_load(2, 2)
        dst_load(2, 2)
        src_wait(0, 0)
        gather(0)
        step(0, 0, do_scatter_wait=False, do_dst_load=False)
        step(1, 1, do_scatter_wait=False, do_dst_load=False)

        def body6(g, carry):
            j0 = 6 * g + 2
            for b in range(6):
                step(j0 + b, (2 + b) % 3)
            return carry

        lax.fori_loop(0, (_NCH - 6) // 6, body6, 0)

        for jj in range(2 + 6 * ((_NCH - 6) // 6), _NCH):
            step(jj, jj % 3,
                 do_dst_load=jj + 1 < _NCH,
                 do_gather_next=jj + 1 < _NCH,
                 do_src_load=jj + 3 < _NCH)
        scatter_wait((_NCH - 2) % 3)
        scatter_wait((_NCH - 1) % 3)

    @pl.when(c == 0)
    def _():
        run(h0_hbm)

    @pl.when(c == 1)
    def _():
        run(h1_hbm)

    plsc.subcore_barrier()
    pltpu.sync_copy(acc.at[pl.ds(s * _ZROWS, _ZROWS)],
                    out_hbm.at[c, pl.ds(s * _ZROWS, _ZROWS)])


_RB = 1000


def _mlp_body(eps_ref, h_ref, n0_ref, n1_ref, w1_ref, b1_ref, w2_ref,
              b2_ref, o_ref):
    neigh = jnp.concatenate([n0_ref[0], n1_ref[0]], axis=1)
    z = (1.0 + eps_ref[0, 0]) * h_ref[...] + neigh
    hid = jnp.maximum(
        jnp.dot(z, w1_ref[...], preferred_element_type=jnp.float32)
        + b1_ref[...], 0.0)
    o_ref[...] = jnp.maximum(
        jnp.dot(hid, w2_ref[...], preferred_element_type=jnp.float32)
        + b2_ref[...], 0.0)


_mlp = pl.pallas_call(
    _mlp_body,
    grid=(_N // _RB,),
    in_specs=[
        pl.BlockSpec(memory_space=pltpu.SMEM),
        pl.BlockSpec((_RB, _D), lambda i: (i, 0)),
        pl.BlockSpec((1, _RB, _DH), lambda i: (0, i, 0)),
        pl.BlockSpec((1, _RB, _DH), lambda i: (1, i, 0)),
        pl.BlockSpec((_D, _D), lambda i: (0, 0)),
        pl.BlockSpec((1, _D), lambda i: (0, 0)),
        pl.BlockSpec((_D, _D), lambda i: (0, 0)),
        pl.BlockSpec((1, _D), lambda i: (0, 0)),
    ],
    out_specs=pl.BlockSpec((_RB, _D), lambda i: (i, 0)),
    out_shape=jax.ShapeDtypeStruct((_N, _D), jnp.float32),
)


def kernel(h, edge_index, eps, W1, b1, W2, b2):
    ei = edge_index
    if ei.dtype != jnp.int32:
        ei = ei.astype(jnp.int32)
    src = ei[1]
    dst = ei[0]
    h0 = h[:, :_DH]
    h1 = h[:, _DH:]
    zeros = jnp.zeros((_ZROWS, _DH), jnp.float32)

    neigh = _sc_neigh(src, dst, h0, h1, zeros)

    eps2d = eps.astype(jnp.float32).reshape(1, 1)
    return _mlp(eps2d, h, neigh, neigh, W1,
                b1.reshape(1, _D), W2, b2.reshape(1, _D))


# speedup vs baseline: 1.2020x; 1.2020x over previous
import functools

import jax
import jax.numpy as jnp
from jax import lax
from jax.experimental import pallas as pl
from jax.experimental.pallas import tpu as pltpu
from jax.experimental.pallas import tpu_sc as plsc

_N = 10000
_E = 160000
_D = 256
_DH = _D // 2

_NT = 16
_CH = 128
_NCH = 78
_TAIL = 16
_EPT = 10000

_ACC_ROWS = 10112
_ZROWS = _ACC_ROWS // _NT

_mesh = plsc.VectorSubcoreMesh(core_axis_name="c", subcore_axis_name="s")


@functools.partial(
    pl.kernel,
    out_type=jax.ShapeDtypeStruct((2, _ACC_ROWS, _DH), jnp.float32),
    mesh=_mesh,
    scratch_types=[
        pltpu.VMEM((_CH,), jnp.int32),
        pltpu.VMEM((_CH,), jnp.int32),
        pltpu.VMEM((_CH,), jnp.int32),
        pltpu.VMEM((_CH,), jnp.int32),
        pltpu.VMEM((_CH,), jnp.int32),
        pltpu.VMEM((_CH,), jnp.int32),
        pltpu.VMEM((_TAIL,), jnp.int32),
        pltpu.VMEM((_TAIL,), jnp.int32),
        pltpu.VMEM((_CH, _DH), jnp.float32),
        pltpu.VMEM((_CH, _DH), jnp.float32),
        pltpu.VMEM((_CH, _DH), jnp.float32),
        pltpu.VMEM_SHARED((_ACC_ROWS, _DH), jnp.float32),
        pltpu.SemaphoreType.DMA,
        pltpu.SemaphoreType.DMA,
        pltpu.SemaphoreType.DMA,
        pltpu.SemaphoreType.DMA,
        pltpu.SemaphoreType.DMA,
        pltpu.SemaphoreType.DMA,
        pltpu.SemaphoreType.DMA,
        pltpu.SemaphoreType.DMA,
        pltpu.SemaphoreType.DMA,
        pltpu.SemaphoreType.DMA,
        pltpu.SemaphoreType.DMA,
        pltpu.SemaphoreType.DMA,
    ],
)
def _sc_neigh(src_hbm, dst_hbm, h0_hbm, h1_hbm, zeros_hbm, out_hbm,
              srcA, srcB, srcC, dstA, dstB, dstC, srcT, dstT,
              rows0, rows1, rows2, acc,
              semSrcA, semSrcB, semSrcC, semDstA, semDstB, semDstC,
              semG0, semG1, semG2, semS0, semS1, semS2):
    c = lax.axis_index("c")
    s = lax.axis_index("s")
    srcI = (srcA, srcB, srcC)
    dstI = (dstA, dstB, dstC)
    semSrc = (semSrcA, semSrcB, semSrcC)
    semDst = (semDstA, semDstB, semDstC)
    rowsR = (rows0, rows1, rows2)
    semG = (semG0, semG1, semG2)
    semS = (semS0, semS1, semS2)

    pltpu.sync_copy(zeros_hbm, acc.at[pl.ds(s * _ZROWS, _ZROWS)])
    plsc.subcore_barrier()

    def src_load(j, k):
        pltpu.async_copy(src_hbm.at[pl.ds(s * _EPT + j * _CH, _CH)],
                         srcI[k], semSrc[k])

    def src_wait(j, k):
        pltpu.make_async_copy(src_hbm.at[pl.ds(s * _EPT + j * _CH, _CH)],
                              srcI[k], semSrc[k]).wait()

    def dst_load(j, k):
        pltpu.async_copy(dst_hbm.at[pl.ds(s * _EPT + j * _CH, _CH)],
                         dstI[k], semDst[k])

    def dst_wait(j, k):
        pltpu.make_async_copy(dst_hbm.at[pl.ds(s * _EPT + j * _CH, _CH)],
                              dstI[k], semDst[k]).wait()

    def run(h_ref):
        def gather(k):
            pltpu.async_copy(h_ref.at[srcI[k]], rowsR[k], semG[k])

        def gather_wait(k):
            pltpu.make_async_copy(h_ref.at[srcI[k]], rowsR[k], semG[k]).wait()

        def scatter(k):
            pltpu.async_copy(rowsR[k], acc.at[dstI[k]], semS[k], add=True)

        def scatter_wait(k):
            pltpu.make_async_copy(rowsR[k], acc.at[dstI[k]], semS[k]).wait()

        def step(jj, k, do_scatter_wait=True, do_dst_load=True,
                 do_gather_next=True, do_src_load=True):
            kn = (k + 1) % 3
            if do_scatter_wait:
                scatter_wait(kn)
            if do_dst_load:
                dst_load(jj + 1, kn)
            if do_gather_next:
                src_wait(jj + 1, kn)
                gather(kn)
            gather_wait(k)
            if do_src_load:
                src_load(jj + 3, k)
            dst_wait(jj, k)
            scatter(k)

        src_load(0, 0)
        dst_load(0, 0)
        src_load(1, 1)
        dst_load(1, 1)
        src_load(2, 2)
        dst_load(2, 2)
        src_wait(0, 0)
        gather(0)
        step(0, 0, do_scatter_wait=False, do_dst_load=False)
        step(1, 1, do_scatter_wait=False, do_dst_load=False)

        def body6(g, carry):
            j0 = 6 * g + 2
            for b in range(6):
                step(j0 + b, (2 + b) % 3)
            return carry

        lax.fori_loop(0, (_NCH - 6) // 6, body6, 0)

        for jj in range(2 + 6 * ((_NCH - 6) // 6), _NCH):
            step(jj, jj % 3,
                 do_dst_load=jj + 1 < _NCH,
                 do_gather_next=jj + 1 < _NCH,
                 do_src_load=jj + 3 < _NCH)
        scatter_wait((_NCH - 2) % 3)
        scatter_wait((_NCH - 1) % 3)

        base = s * _EPT + _NCH * _CH
        pltpu.sync_copy(src_hbm.at[pl.ds(base, _TAIL)], srcT)
        pltpu.sync_copy(dst_hbm.at[pl.ds(base, _TAIL)], dstT)
        pltpu.async_copy(h_ref.at[srcT], rowsR[0].at[pl.ds(0, _TAIL)],
                         semG[0])
        pltpu.make_async_copy(h_ref.at[srcT], rowsR[0].at[pl.ds(0, _TAIL)],
                              semG[0]).wait()
        pltpu.sync_copy(rowsR[0].at[pl.ds(0, _TAIL)], acc.at[dstT], add=True)

    @pl.when(c == 0)
    def _():
        run(h0_hbm)

    @pl.when(c == 1)
    def _():
        run(h1_hbm)

    plsc.subcore_barrier()
    pltpu.sync_copy(acc.at[pl.ds(s * _ZROWS, _ZROWS)],
                    out_hbm.at[c, pl.ds(s * _ZROWS, _ZROWS)])


_RB = 1000


def _mlp_body(eps_ref, h_ref, n0_ref, n1_ref, w1_ref, b1_ref, w2_ref,
              b2_ref, o_ref):
    neigh = jnp.concatenate([n0_ref[0], n1_ref[0]], axis=1)
    z = (1.0 + eps_ref[0, 0]) * h_ref[...] + neigh
    hid = jnp.maximum(
        jnp.dot(z, w1_ref[...], preferred_element_type=jnp.float32)
        + b1_ref[...], 0.0)
    o_ref[...] = jnp.maximum(
        jnp.dot(hid, w2_ref[...], preferred_element_type=jnp.float32)
        + b2_ref[...], 0.0)


_mlp = pl.pallas_call(
    _mlp_body,
    grid=(_N // _RB,),
    in_specs=[
        pl.BlockSpec(memory_space=pltpu.SMEM),
        pl.BlockSpec((_RB, _D), lambda i: (i, 0)),
        pl.BlockSpec((1, _RB, _DH), lambda i: (0, i, 0)),
        pl.BlockSpec((1, _RB, _DH), lambda i: (1, i, 0)),
        pl.BlockSpec((_D, _D), lambda i: (0, 0)),
        pl.BlockSpec((1, _D), lambda i: (0, 0)),
        pl.BlockSpec((_D, _D), lambda i: (0, 0)),
        pl.BlockSpec((1, _D), lambda i: (0, 0)),
    ],
    out_specs=pl.BlockSpec((_RB, _D), lambda i: (i, 0)),
    out_shape=jax.ShapeDtypeStruct((_N, _D), jnp.float32),
)


def kernel(h, edge_index, eps, W1, b1, W2, b2):
    ei = edge_index
    if ei.dtype != jnp.int32:
        ei = ei.astype(jnp.int32)
    src = ei[1]
    dst = ei[0]
    h0 = h[:, :_DH]
    h1 = h[:, _DH:]
    zeros = jnp.zeros((_ZROWS, _DH), jnp.float32)

    neigh = _sc_neigh(src, dst, h0, h1, zeros)

    eps2d = eps.astype(jnp.float32).reshape(1, 1)
    return _mlp(eps2d, h, neigh, neigh, W1,
                b1.reshape(1, _D), W2, b2.reshape(1, _D))


# speedup vs baseline: 1.2384x; 1.0303x over previous
import functools

import jax
import jax.numpy as jnp
from jax import lax
from jax.experimental import pallas as pl
from jax.experimental.pallas import tpu as pltpu
from jax.experimental.pallas import tpu_sc as plsc

_N = 10000
_E = 160000
_D = 256
_DH = _D // 2

_NT = 16
_CH = 80
_NCH = 125
_EPT = _NCH * _CH

_ACC_ROWS = 10112
_ZROWS = _ACC_ROWS // _NT

_mesh = plsc.VectorSubcoreMesh(core_axis_name="c", subcore_axis_name="s")


@functools.partial(
    pl.kernel,
    out_type=jax.ShapeDtypeStruct((2, _ACC_ROWS, _DH), jnp.float32),
    mesh=_mesh,
    scratch_types=[
        pltpu.VMEM((_CH,), jnp.int32),
        pltpu.VMEM((_CH,), jnp.int32),
        pltpu.VMEM((_CH,), jnp.int32),
        pltpu.VMEM((_CH,), jnp.int32),
        pltpu.VMEM((_CH,), jnp.int32),
        pltpu.VMEM((_CH,), jnp.int32),
        pltpu.VMEM((_CH, _DH), jnp.float32),
        pltpu.VMEM((_CH, _DH), jnp.float32),
        pltpu.VMEM((_CH, _DH), jnp.float32),
        pltpu.VMEM_SHARED((_ACC_ROWS, _DH), jnp.float32),
        pltpu.SemaphoreType.DMA,
        pltpu.SemaphoreType.DMA,
        pltpu.SemaphoreType.DMA,
        pltpu.SemaphoreType.DMA,
        pltpu.SemaphoreType.DMA,
        pltpu.SemaphoreType.DMA,
        pltpu.SemaphoreType.DMA,
        pltpu.SemaphoreType.DMA,
        pltpu.SemaphoreType.DMA,
        pltpu.SemaphoreType.DMA,
        pltpu.SemaphoreType.DMA,
        pltpu.SemaphoreType.DMA,
    ],
)
def _sc_neigh(src_hbm, dst_hbm, h0_hbm, h1_hbm, zeros_hbm, out_hbm,
              srcA, srcB, srcC, dstA, dstB, dstC, rows0, rows1, rows2, acc,
              semSrcA, semSrcB, semSrcC, semDstA, semDstB, semDstC,
              semG0, semG1, semG2, semS0, semS1, semS2):
    c = lax.axis_index("c")
    s = lax.axis_index("s")
    srcI = (srcA, srcB, srcC)
    dstI = (dstA, dstB, dstC)
    semSrc = (semSrcA, semSrcB, semSrcC)
    semDst = (semDstA, semDstB, semDstC)
    rowsR = (rows0, rows1, rows2)
    semG = (semG0, semG1, semG2)
    semS = (semS0, semS1, semS2)

    pltpu.sync_copy(zeros_hbm, acc.at[pl.ds(s * _ZROWS, _ZROWS)])
    plsc.subcore_barrier()

    def src_load(j, k):
        pltpu.async_copy(src_hbm.at[pl.ds(s * _EPT + j * _CH, _CH)],
                         srcI[k], semSrc[k])

    def src_wait(j, k):
        pltpu.make_async_copy(src_hbm.at[pl.ds(s * _EPT + j * _CH, _CH)],
                              srcI[k], semSrc[k]).wait()

    def dst_load(j, k):
        pltpu.async_copy(dst_hbm.at[pl.ds(s * _EPT + j * _CH, _CH)],
                         dstI[k], semDst[k])

    def dst_wait(j, k):
        pltpu.make_async_copy(dst_hbm.at[pl.ds(s * _EPT + j * _CH, _CH)],
                              dstI[k], semDst[k]).wait()

    def run(h_ref):
        def gather(k):
            pltpu.async_copy(h_ref.at[srcI[k]], rowsR[k], semG[k])

        def gather_wait(k):
            pltpu.make_async_copy(h_ref.at[srcI[k]], rowsR[k], semG[k]).wait()

        def scatter(k):
            pltpu.async_copy(rowsR[k], acc.at[dstI[k]], semS[k], add=True)

        def scatter_wait(k):
            pltpu.make_async_copy(rowsR[k], acc.at[dstI[k]], semS[k]).wait()

        def step(jj, k, do_scatter_wait=True, do_dst_load=True,
                 do_gather_next=True, do_src_load=True):
            kn = (k + 1) % 3
            if do_scatter_wait:
                scatter_wait(kn)
            if do_dst_load:
                dst_load(jj + 1, kn)
            if do_gather_next:
                src_wait(jj + 1, kn)
                gather(kn)
            gather_wait(k)
            if do_src_load:
                src_load(jj + 3, k)
            dst_wait(jj, k)
            scatter(k)

        src_load(0, 0)
        dst_load(0, 0)
        src_load(1, 1)
        dst_load(1, 1)
        src_load(2, 2)
        dst_load(2, 2)
        src_wait(0, 0)
        gather(0)
        step(0, 0, do_scatter_wait=False, do_dst_load=False)
        step(1, 1, do_scatter_wait=False, do_dst_load=False)

        def body6(g, carry):
            j0 = 6 * g + 2
            for b in range(6):
                step(j0 + b, (2 + b) % 3)
            return carry

        lax.fori_loop(0, (_NCH - 6) // 6, body6, 0)

        for jj in range(2 + 6 * ((_NCH - 6) // 6), _NCH):
            step(jj, jj % 3,
                 do_dst_load=jj + 1 < _NCH,
                 do_gather_next=jj + 1 < _NCH,
                 do_src_load=jj + 3 < _NCH)
        scatter_wait((_NCH - 2) % 3)
        scatter_wait((_NCH - 1) % 3)

    @pl.when(c == 0)
    def _():
        run(h0_hbm)

    @pl.when(c == 1)
    def _():
        run(h1_hbm)

    plsc.subcore_barrier()
    pltpu.sync_copy(acc.at[pl.ds(s * _ZROWS, _ZROWS)],
                    out_hbm.at[c, pl.ds(s * _ZROWS, _ZROWS)])


_RB = 1000


def _mlp_body(eps_ref, h_ref, n0_ref, n1_ref, w1_ref, b1_ref, w2_ref,
              b2_ref, o_ref):
    neigh = jnp.concatenate([n0_ref[0], n1_ref[0]], axis=1)
    z = (1.0 + eps_ref[0, 0]) * h_ref[...] + neigh
    hid = jnp.maximum(
        jnp.dot(z, w1_ref[...], preferred_element_type=jnp.float32)
        + b1_ref[...], 0.0)
    o_ref[...] = jnp.maximum(
        jnp.dot(hid, w2_ref[...], preferred_element_type=jnp.float32)
        + b2_ref[...], 0.0)


_mlp = pl.pallas_call(
    _mlp_body,
    grid=(_N // _RB,),
    in_specs=[
        pl.BlockSpec(memory_space=pltpu.SMEM),
        pl.BlockSpec((_RB, _D), lambda i: (i, 0)),
        pl.BlockSpec((1, _RB, _DH), lambda i: (0, i, 0)),
        pl.BlockSpec((1, _RB, _DH), lambda i: (1, i, 0)),
        pl.BlockSpec((_D, _D), lambda i: (0, 0)),
        pl.BlockSpec((1, _D), lambda i: (0, 0)),
        pl.BlockSpec((_D, _D), lambda i: (0, 0)),
        pl.BlockSpec((1, _D), lambda i: (0, 0)),
    ],
    out_specs=pl.BlockSpec((_RB, _D), lambda i: (i, 0)),
    out_shape=jax.ShapeDtypeStruct((_N, _D), jnp.float32),
)


def kernel(h, edge_index, eps, W1, b1, W2, b2):
    ei = edge_index
    if ei.dtype != jnp.int32:
        ei = ei.astype(jnp.int32)
    src = ei[1]
    dst = ei[0]
    h0 = h[:, :_DH]
    h1 = h[:, _DH:]
    zeros = jnp.zeros((_ZROWS, _DH), jnp.float32)

    neigh = _sc_neigh(src, dst, h0, h1, zeros)

    eps2d = eps.astype(jnp.float32).reshape(1, 1)
    return _mlp(eps2d, h, neigh, neigh, W1,
                b1.reshape(1, _D), W2, b2.reshape(1, _D))


# speedup vs baseline: 1.2557x; 1.0140x over previous
import functools

import jax
import jax.numpy as jnp
from jax import lax
from jax.experimental import pallas as pl
from jax.experimental.pallas import tpu as pltpu
from jax.experimental.pallas import tpu_sc as plsc

_N = 10000
_E = 160000
_D = 256
_DH = _D // 2

_NT = 16
_CH = 80
_NCH = 125
_EPT = _NCH * _CH

_ACC_ROWS = 10112
_ZROWS = _ACC_ROWS // _NT

_mesh = plsc.VectorSubcoreMesh(core_axis_name="c", subcore_axis_name="s")


@functools.partial(
    pl.kernel,
    out_type=jax.ShapeDtypeStruct((2, _ACC_ROWS, _DH), jnp.float32),
    mesh=_mesh,
    scratch_types=[
        pltpu.VMEM((_CH,), jnp.int32),
        pltpu.VMEM((_CH,), jnp.int32),
        pltpu.VMEM((_CH,), jnp.int32),
        pltpu.VMEM((_CH,), jnp.int32),
        pltpu.VMEM((_CH,), jnp.int32),
        pltpu.VMEM((_CH,), jnp.int32),
        pltpu.VMEM((_CH, _DH), jnp.float32),
        pltpu.VMEM((_CH, _DH), jnp.float32),
        pltpu.VMEM((_CH, _DH), jnp.float32),
        pltpu.VMEM_SHARED((_ACC_ROWS, _DH), jnp.float32),
        pltpu.SemaphoreType.DMA,
        pltpu.SemaphoreType.DMA,
        pltpu.SemaphoreType.DMA,
        pltpu.SemaphoreType.DMA,
        pltpu.SemaphoreType.DMA,
        pltpu.SemaphoreType.DMA,
        pltpu.SemaphoreType.DMA,
        pltpu.SemaphoreType.DMA,
        pltpu.SemaphoreType.DMA,
        pltpu.SemaphoreType.DMA,
        pltpu.SemaphoreType.DMA,
        pltpu.SemaphoreType.DMA,
    ],
)
def _sc_neigh(src_hbm, dst_hbm, h0_hbm, h1_hbm, zeros_hbm, out_hbm,
              srcA, srcB, srcC, dstA, dstB, dstC, rows0, rows1, rows2, acc,
              semSrcA, semSrcB, semSrcC, semDstA, semDstB, semDstC,
              semG0, semG1, semG2, semS0, semS1, semS2):
    c = lax.axis_index("c")
    s = lax.axis_index("s")
    srcI = (srcA, srcB, srcC)
    dstI = (dstA, dstB, dstC)
    semSrc = (semSrcA, semSrcB, semSrcC)
    semDst = (semDstA, semDstB, semDstC)
    rowsR = (rows0, rows1, rows2)
    semG = (semG0, semG1, semG2)
    semS = (semS0, semS1, semS2)

    def src_load(j, k):
        pltpu.async_copy(src_hbm.at[pl.ds(s * _EPT + j * _CH, _CH)],
                         srcI[k], semSrc[k])

    def src_wait(j, k):
        pltpu.make_async_copy(src_hbm.at[pl.ds(s * _EPT + j * _CH, _CH)],
                              srcI[k], semSrc[k]).wait()

    def dst_load(j, k):
        pltpu.async_copy(dst_hbm.at[pl.ds(s * _EPT + j * _CH, _CH)],
                         dstI[k], semDst[k])

    def dst_wait(j, k):
        pltpu.make_async_copy(dst_hbm.at[pl.ds(s * _EPT + j * _CH, _CH)],
                              dstI[k], semDst[k]).wait()

    def run(h_ref):
        def gather(k):
            pltpu.async_copy(h_ref.at[srcI[k]], rowsR[k], semG[k])

        def gather_wait(k):
            pltpu.make_async_copy(h_ref.at[srcI[k]], rowsR[k], semG[k]).wait()

        def scatter(k):
            pltpu.async_copy(rowsR[k], acc.at[dstI[k]], semS[k], add=True)

        def scatter_wait(k):
            pltpu.make_async_copy(rowsR[k], acc.at[dstI[k]], semS[k]).wait()

        def step(jj, k, do_scatter_wait=True, do_dst_load=True,
                 do_gather_next=True, do_src_load=True):
            kn = (k + 1) % 3
            if do_scatter_wait:
                scatter_wait(kn)
            if do_dst_load:
                dst_load(jj + 1, kn)
            if do_gather_next:
                src_wait(jj + 1, kn)
                gather(kn)
            gather_wait(k)
            if do_src_load:
                src_load(jj + 3, k)
            dst_wait(jj, k)
            scatter(k)

        src_load(0, 0)
        dst_load(0, 0)
        src_load(1, 1)
        dst_load(1, 1)
        src_load(2, 2)
        dst_load(2, 2)
        src_wait(0, 0)
        gather(0)
        pltpu.sync_copy(zeros_hbm, acc.at[pl.ds(s * _ZROWS, _ZROWS)])
        plsc.subcore_barrier()
        step(0, 0, do_scatter_wait=False, do_dst_load=False)
        step(1, 1, do_scatter_wait=False, do_dst_load=False)

        def body6(g, carry):
            j0 = 6 * g + 2
            for b in range(6):
                step(j0 + b, (2 + b) % 3)
            return carry

        lax.fori_loop(0, (_NCH - 6) // 6, body6, 0)

        for jj in range(2 + 6 * ((_NCH - 6) // 6), _NCH):
            step(jj, jj % 3,
                 do_dst_load=jj + 1 < _NCH,
                 do_gather_next=jj + 1 < _NCH,
                 do_src_load=jj + 3 < _NCH)
        scatter_wait((_NCH - 2) % 3)
        scatter_wait((_NCH - 1) % 3)

    @pl.when(c == 0)
    def _():
        run(h0_hbm)

    @pl.when(c == 1)
    def _():
        run(h1_hbm)

    plsc.subcore_barrier()
    pltpu.sync_copy(acc.at[pl.ds(s * _ZROWS, _ZROWS)],
                    out_hbm.at[c, pl.ds(s * _ZROWS, _ZROWS)])


_RB = 2000


def _mlp_body(eps_ref, h_ref, n0_ref, n1_ref, w1_ref, b1_ref, w2_ref,
              b2_ref, o_ref):
    neigh = jnp.concatenate([n0_ref[0], n1_ref[0]], axis=1)
    z = (1.0 + eps_ref[0, 0]) * h_ref[...] + neigh
    hid = jnp.maximum(
        jnp.dot(z, w1_ref[...], preferred_element_type=jnp.float32)
        + b1_ref[...], 0.0)
    o_ref[...] = jnp.maximum(
        jnp.dot(hid, w2_ref[...], preferred_element_type=jnp.float32)
        + b2_ref[...], 0.0)


_mlp = pl.pallas_call(
    _mlp_body,
    grid=(_N // _RB,),
    in_specs=[
        pl.BlockSpec(memory_space=pltpu.SMEM),
        pl.BlockSpec((_RB, _D), lambda i: (i, 0)),
        pl.BlockSpec((1, _RB, _DH), lambda i: (0, i, 0)),
        pl.BlockSpec((1, _RB, _DH), lambda i: (1, i, 0)),
        pl.BlockSpec((_D, _D), lambda i: (0, 0)),
        pl.BlockSpec((1, _D), lambda i: (0, 0)),
        pl.BlockSpec((_D, _D), lambda i: (0, 0)),
        pl.BlockSpec((1, _D), lambda i: (0, 0)),
    ],
    out_specs=pl.BlockSpec((_RB, _D), lambda i: (i, 0)),
    out_shape=jax.ShapeDtypeStruct((_N, _D), jnp.float32),
)


def kernel(h, edge_index, eps, W1, b1, W2, b2):
    ei = edge_index
    if ei.dtype != jnp.int32:
        ei = ei.astype(jnp.int32)
    src = ei[1]
    dst = ei[0]
    h0 = h[:, :_DH]
    h1 = h[:, _DH:]
    zeros = jnp.zeros((_ZROWS, _DH), jnp.float32)

    neigh = _sc_neigh(src, dst, h0, h1, zeros)

    eps2d = eps.astype(jnp.float32).reshape(1, 1)
    return _mlp(eps2d, h, neigh, neigh, W1,
                b1.reshape(1, _D), W2, b2.reshape(1, _D))
